# baseline (speedup 1.0000x reference)
import functools

import jax
import jax.numpy as jnp
from jax import lax
from jax.experimental import pallas as pl
from jax.experimental.pallas import tpu as pltpu
from jax.experimental.pallas import tpu_sc as plsc

N, G, T, S = 320000, 128, 16, 256
SCALE = 1000000.0

NC, NS = 2, 16
NW = NC * NS
RW = N // NW
CHUNK = 80
NCH = RW // CHUNK


def _sc_segment_sum():
    mesh = plsc.VectorSubcoreMesh(core_axis_name="c", subcore_axis_name="s")

    @functools.partial(
        pl.kernel,
        mesh=mesh,
        out_type=jax.ShapeDtypeStruct((NC, S, G), jnp.float32),
        scratch_types=[
            pltpu.VMEM((CHUNK, G), jnp.float32),
            pltpu.VMEM((CHUNK, G), jnp.float32),
            pltpu.VMEM((CHUNK,), jnp.int32),
            pltpu.VMEM((CHUNK,), jnp.int32),
            pltpu.VMEM((16, G), jnp.float32),
            pltpu.VMEM_SHARED((S, G), jnp.float32),
            pltpu.SemaphoreType.DMA,
            pltpu.SemaphoreType.DMA,
            pltpu.SemaphoreType.DMA,
            pltpu.SemaphoreType.DMA,
        ],
    )
    def seg_sum(x_hbm, idx_hbm, out_hbm,
                x_v0, x_v1, i_v0, i_v1, z_v, acc_sh,
                sx0, sx1, si0, si1):
        cid = lax.axis_index("c")
        sid = lax.axis_index("s")
        wid = cid * NS + sid
        base = wid * RW

        zrow = jnp.zeros((16,), jnp.float32)
        for r in range(16):
            for c8 in range(G // 16):
                z_v[r, pl.ds(c8 * 16, 16)] = zrow
        pltpu.sync_copy(z_v, acc_sh.at[pl.ds(sid * 16, 16)])
        plsc.subcore_barrier()

        def start(ch, x_v, i_v, sx, si):
            cpx = pltpu.make_async_copy(
                x_hbm.at[pl.ds(base + ch * CHUNK, CHUNK)], x_v, sx)
            cpx.start()
            cpi = pltpu.make_async_copy(idx_hbm.at[wid, ch], i_v, si)
            cpi.start()

        def wait(x_v, i_v, sx, si):
            pltpu.make_async_copy(x_hbm.at[pl.ds(0, CHUNK)], x_v, sx).wait()
            pltpu.make_async_copy(idx_hbm.at[0, 0], i_v, si).wait()

        def flush(x_v, i_v):
            pltpu.sync_copy(x_v, acc_sh.at[i_v], add=True)

        start(0, x_v0, i_v0, sx0, si0)
        start(1, x_v1, i_v1, sx1, si1)

        def body(j, carry):
            c0 = 2 * j
            wait(x_v0, i_v0, sx0, si0)
            flush(x_v0, i_v0)
            start(c0 + 2, x_v0, i_v0, sx0, si0)
            wait(x_v1, i_v1, sx1, si1)
            flush(x_v1, i_v1)
            start(jnp.minimum(c0 + 3, NCH - 1), x_v1, i_v1, sx1, si1)
            return carry

        lax.fori_loop(0, (NCH - 1) // 2, body, 0)
        wait(x_v0, i_v0, sx0, si0)
        flush(x_v0, i_v0)
        wait(x_v1, i_v1, sx1, si1)

        plsc.subcore_barrier()
        pltpu.sync_copy(acc_sh.at[pl.ds(sid * 16, 16)],
                        out_hbm.at[cid, pl.ds(sid * 16, 16)])

    return seg_sum


def _tc_finish(p_ref, w_ref, ilr_ref, xb_ref):
    raw = p_ref[0] + p_ref[1]
    rs = jnp.sum(raw, axis=1, keepdims=True)
    xb = raw * (SCALE / jnp.clip(rs, 1e-12, None))
    xb_ref[...] = xb
    ilr_ref[...] = jax.lax.dot_general(
        xb, w_ref[...], (((1,), (1,)), ((), ())),
        preferred_element_type=jnp.float32)


_tc_finish_call = pl.pallas_call(
    _tc_finish,
    out_shape=[
        jax.ShapeDtypeStruct((S, T), jnp.float32),
        jax.ShapeDtypeStruct((S, G), jnp.float32),
    ],
)


def kernel(X_batch, batch_idx, W):
    ids3 = batch_idx.astype(jnp.int32).reshape(NW, NCH, CHUNK)
    partials = _sc_segment_sum()(X_batch, ids3)
    ilr_y, X_bulk = _tc_finish_call(partials, W)
    return (ilr_y, X_bulk)
